# baseline (speedup 1.0000x reference)
import jax
import jax.numpy as jnp
from jax.experimental import pallas as pl
from jax.experimental.pallas import tpu as pltpu
from jax.experimental.pallas import tpu_sc as plsc


def _gather_body(x_hbm, out_hbm, sem0, sem1):
    c0 = pltpu.make_async_copy(
        x_hbm.at[pl.ds(2, 1)], out_hbm.at[pl.ds(0, 1)], sem0
    )
    c1 = pltpu.make_async_copy(
        x_hbm.at[pl.ds(4, 2)], out_hbm.at[pl.ds(1, 2)], sem1
    )
    c0.start()
    c1.start()
    c0.wait()
    c1.wait()


@jax.jit
def kernel(x):
    mesh = plsc.ScalarSubcoreMesh(axis_name="c", num_cores=1)
    run = pl.kernel(
        _gather_body,
        mesh=mesh,
        out_type=jax.ShapeDtypeStruct((3, 128), jnp.float32),
        scratch_types=[pltpu.SemaphoreType.DMA, pltpu.SemaphoreType.DMA],
    )
    return run(x)


# speedup vs baseline: 1.0008x; 1.0008x over previous
import jax
import jax.numpy as jnp
from jax.experimental import pallas as pl
from jax.experimental.pallas import tpu as pltpu
from jax.experimental.pallas import tpu_sc as plsc


def _gather_body(x_hbm, out_hbm, sem0):
    c0 = pltpu.make_async_copy(
        x_hbm.at[pl.ds(2, 1)], out_hbm.at[pl.ds(0, 1)], sem0
    )
    c1 = pltpu.make_async_copy(
        x_hbm.at[pl.ds(4, 2)], out_hbm.at[pl.ds(1, 2)], sem0
    )
    c0.start()
    c1.start()
    c0.wait()
    c1.wait()


@jax.jit
def kernel(x):
    mesh = plsc.ScalarSubcoreMesh(axis_name="c", num_cores=1)
    run = pl.kernel(
        _gather_body,
        mesh=mesh,
        out_type=jax.ShapeDtypeStruct((3, 128), jnp.float32),
        scratch_types=[pltpu.SemaphoreType.DMA],
    )
    return run(x)
